# baseline (speedup 1.0000x reference)
import functools

import jax
import jax.numpy as jnp
from jax import lax
from jax.experimental import pallas as pl
from jax.experimental.pallas import tpu as pltpu
from jax.experimental.pallas import tpu_sc as plsc

_ROWS = 128
_COLS = 100000
_SUP_STRIDE = 200
_W = 6144
_NFULL = 16
_SC_COLS = _NFULL * _W
_TAIL = _COLS - _SC_COLS
_GROUP_ROWS = 8

_mesh = plsc.VectorSubcoreMesh(core_axis_name="c", subcore_axis_name="s")


@functools.partial(
    pl.kernel,
    out_type=jax.ShapeDtypeStruct((_ROWS, _COLS), jnp.float32),
    mesh=_mesh,
    scratch_types=[
        pltpu.VMEM((2, _GROUP_ROWS, _W), jnp.float32),
        pltpu.SemaphoreType.DMA((2,)),
        pltpu.SemaphoreType.DMA((2,)),
    ],
)
def _sc_kernel(x_hbm, o_hbm, bufs, sem_in, sem_out):
    wid = lax.axis_index("s") * 2 + lax.axis_index("c")
    group = wid // 2
    half = wid % 2
    r0 = group * _GROUP_ROWS
    col0 = half * 8 * _W

    lane = lax.iota(jnp.int32, 16)
    _chunk_ids = {0: list(range(0, _W, _SUP_STRIDE)), 1: [6200 - _W]}

    def _mask_window(buf, r, c):
        c0 = c - (c % 16)
        v = buf[r, pl.ds(c0, 16)]
        buf[r, pl.ds(c0, 16)] = jnp.where(lane == (c % 16), -jnp.inf, v)

    for k in range(8):
        b = k % 2
        c = col0 + k * _W
        if k >= 2:
            pltpu.make_async_copy(
                bufs.at[b],
                o_hbm.at[pl.ds(r0, _GROUP_ROWS), pl.ds(0, _W)],
                sem_out.at[b],
            ).wait()
        pltpu.async_copy(
            x_hbm.at[pl.ds(r0, _GROUP_ROWS), pl.ds(c, _W)],
            bufs.at[b],
            sem_in.at[b],
        ).wait()
        if k in (0, 1):
            @pl.when(half == 0)
            def _mask(b=b, k=k):
                for r in range(_GROUP_ROWS):
                    for cc in _chunk_ids[k]:
                        _mask_window(bufs.at[b], r, cc)
        pltpu.async_copy(
            bufs.at[b],
            o_hbm.at[pl.ds(r0, _GROUP_ROWS), pl.ds(c, _W)],
            sem_out.at[b],
        )

    for b in range(2):
        pltpu.make_async_copy(
            bufs.at[b],
            o_hbm.at[pl.ds(r0, _GROUP_ROWS), pl.ds(0, _W)],
            sem_out.at[b],
        ).wait()


def _tail_body(sc_ref, x_hbm, o_hbm, vbuf, s_in, s_out):
    del sc_ref
    pltpu.make_async_copy(
        x_hbm.at[:, pl.ds(_SC_COLS, _TAIL)], vbuf, s_in
    ).start()
    pltpu.make_async_copy(
        x_hbm.at[:, pl.ds(_SC_COLS, _TAIL)], vbuf, s_in
    ).wait()
    pltpu.make_async_copy(
        vbuf, o_hbm.at[:, pl.ds(_SC_COLS, _TAIL)], s_out
    ).start()
    pltpu.make_async_copy(
        vbuf, o_hbm.at[:, pl.ds(_SC_COLS, _TAIL)], s_out
    ).wait()


def _tail_fix(sc_out, scores):
    return pl.pallas_call(
        _tail_body,
        in_specs=[
            pl.BlockSpec(memory_space=pl.MemorySpace.ANY),
            pl.BlockSpec(memory_space=pl.MemorySpace.ANY),
        ],
        out_specs=pl.BlockSpec(memory_space=pl.MemorySpace.ANY),
        out_shape=jax.ShapeDtypeStruct((_ROWS, _COLS), jnp.float32),
        scratch_shapes=[
            pltpu.MemorySpace.VMEM((_ROWS, _TAIL), jnp.float32),
            pltpu.SemaphoreType.DMA,
            pltpu.SemaphoreType.DMA,
        ],
        input_output_aliases={0: 0},
    )(sc_out, scores)


def kernel(scores):
    return _tail_fix(_sc_kernel(scores), scores)


# speedup vs baseline: 1.6147x; 1.6147x over previous
import jax
import jax.numpy as jnp
from jax.experimental import pallas as pl
from jax.experimental.pallas import tpu as pltpu

_ROWS = 128
_COLS = 100000
_SUP_STRIDE = 200
_SUP_LIMIT = 6400
_N_SUP = _SUP_LIMIT // _SUP_STRIDE
_WIN = 128


def _win_start(k):
    return (k * _SUP_STRIDE) // _WIN * _WIN


def _scatter_body(x_any, o_hbm, wins, sem_in, sem_out):
    del x_any
    for k in range(_N_SUP):
        pltpu.make_async_copy(
            o_hbm.at[:, pl.ds(_win_start(k), _WIN)], wins.at[k], sem_in.at[k]
        ).start()
    neg = jnp.full((_ROWS, 1), -jnp.inf, jnp.float32)
    for k in range(_N_SUP):
        pltpu.make_async_copy(
            o_hbm.at[:, pl.ds(_win_start(k), _WIN)], wins.at[k], sem_in.at[k]
        ).wait()
        off = k * _SUP_STRIDE - _win_start(k)
        wins[k, :, off : off + 1] = neg
        pltpu.make_async_copy(
            wins.at[k], o_hbm.at[:, pl.ds(_win_start(k), _WIN)], sem_out.at[k]
        ).start()
    for k in range(_N_SUP):
        pltpu.make_async_copy(
            wins.at[k], o_hbm.at[:, pl.ds(_win_start(k), _WIN)], sem_out.at[k]
        ).wait()


def kernel(scores):
    return pl.pallas_call(
        _scatter_body,
        in_specs=[pl.BlockSpec(memory_space=pl.MemorySpace.ANY)],
        out_specs=pl.BlockSpec(memory_space=pl.MemorySpace.ANY),
        out_shape=jax.ShapeDtypeStruct((_ROWS, _COLS), scores.dtype),
        scratch_shapes=[
            pltpu.MemorySpace.VMEM((_N_SUP, _ROWS, _WIN), jnp.float32),
            pltpu.SemaphoreType.DMA((_N_SUP,)),
            pltpu.SemaphoreType.DMA((_N_SUP,)),
        ],
        input_output_aliases={0: 0},
    )(scores)
